# baseline (speedup 1.0000x reference)
import functools

import jax
import jax.numpy as jnp
from jax import lax
from jax.experimental import pallas as pl
from jax.experimental.pallas import tpu as pltpu
from jax.experimental.pallas import tpu_sc as plsc

_N = 327680
_D = 16
_ZCH = 1000000
_NC = 2
_NS = 16
_NW = _NC * _NS
_NPW = _N // _NW
_C = 128
_NCH = _NPW // _C
_NBUF = 4
_NGRP = _NCH // _NBUF
_L = 16


def _make_sc_kernel():
  mesh = plsc.VectorSubcoreMesh(core_axis_name="c", subcore_axis_name="s")

  @functools.partial(
      pl.kernel,
      mesh=mesh,
      compiler_params=pltpu.CompilerParams(use_tc_tiling_on_sc=False),
      out_type=[
          jax.ShapeDtypeStruct((_N,), jnp.int32),
          jax.ShapeDtypeStruct((_N,), jnp.int32),
          jax.ShapeDtypeStruct((_NW, _L), jnp.float32),
      ],
      scratch_types=[
          pltpu.VMEM((_NPW,), jnp.int32),
          pltpu.VMEM((_NBUF * _C, _D), jnp.float32),
          pltpu.VMEM((_L,), jnp.float32),
      ] + [pltpu.SemaphoreType.DMA] * _NBUF,
  )
  def sc_kernel(v0_hbm, v1_hbm, t0_hbm, t1_hbm, r0_hbm, r1_hbm, parts_hbm,
                idx_v, rows_v, acc_v, *sems):
    wid = lax.axis_index("s") * _NC + lax.axis_index("c")
    base = wid * _NPW

    def run_feature(vals_hbm, table_hbm, out_hbm, accs):
      pltpu.sync_copy(vals_hbm.at[pl.ds(base, _NPW)], idx_v)

      def mod_body(i, carry):
        v = idx_v[pl.ds(i * _L, _L)]
        v = jnp.where(v >= 2 * _ZCH, v - 2 * _ZCH, v)
        v = jnp.where(v >= _ZCH, v - _ZCH, v)
        idx_v[pl.ds(i * _L, _L)] = v
        return carry

      lax.fori_loop(0, _NPW // _L, mod_body, 0)
      pltpu.sync_copy(idx_v, out_hbm.at[pl.ds(base, _NPW)])

      def group(g, accs):
        handles = []
        for b in range(_NBUF):
          c0 = (g * _NBUF + b) * _C
          handles.append(pltpu.async_copy(
              table_hbm.at[idx_v.at[pl.ds(c0, _C)]],
              rows_v.at[pl.ds(b * _C, _C)],
              sems[b]))
        for b in range(_NBUF):
          handles[b].wait()

          def row_body(r, accs, _b=b):
            a0, a1, a2, a3 = accs
            rb = _b * _C + r * 4
            a0 = a0 + rows_v[rb, :]
            a1 = a1 + rows_v[rb + 1, :]
            a2 = a2 + rows_v[rb + 2, :]
            a3 = a3 + rows_v[rb + 3, :]
            return (a0, a1, a2, a3)

          accs = lax.fori_loop(0, _C // 4, row_body, accs)
        return accs

      return lax.fori_loop(0, _NGRP, group, accs)

    accs = tuple(jnp.zeros((_L,), jnp.float32) for _ in range(4))
    accs = run_feature(v0_hbm, t0_hbm, r0_hbm, accs)
    accs = run_feature(v1_hbm, t1_hbm, r1_hbm, accs)
    acc_v[...] = (accs[0] + accs[1]) + (accs[2] + accs[3])
    pltpu.sync_copy(acc_v, parts_hbm.at[wid])

  return sc_kernel


_SC_KERNEL = _make_sc_kernel()


def kernel(values_0, values_1, table_0, table_1):
  r0, r1, parts = _SC_KERNEL(values_0, values_1, table_0, table_1)
  loss = jnp.sum(parts) / jnp.float32(2 * _N * _D)
  return (loss, r0, r1)
